# Initial kernel scaffold; baseline (speedup 1.0000x reference)
import jax
import jax.numpy as jnp
from jax.experimental import pallas as pl


def kernel(x, edge_index, Wl, Wr, att, bias, ln_gamma, ln_beta):
    raise NotImplementedError("write your pallas kernel here")


# speedup vs baseline: 1.3280x; 1.3280x over previous
import functools

import jax
import jax.numpy as jnp
from jax import lax
from jax.experimental import pallas as pl
from jax.experimental.pallas import tpu as pltpu
from jax.experimental.pallas import tpu_sc as plsc

N = 10000
NP = 10240
D = 256
H = 128
E = 160000
ET = E + N
EP = 172032
NC, NS, L = 2, 16, 16
NB32 = 84
NB16 = 168
BT = 64
HNP = NP // 2
RPT = NP // NS


def _proj_body(x_ref, wl_ref, wr_ref, xl_ref, xr_ref):
    xb = x_ref[...]
    xl_ref[...] = jnp.dot(xb, wl_ref[...], preferred_element_type=jnp.float32)
    xr_ref[...] = jnp.dot(xb, wr_ref[...], preferred_element_type=jnp.float32)


def _project(xpad, Wl, Wr):
    blk = 1024
    nb = NP // blk
    return pl.pallas_call(
        _proj_body,
        grid=(nb, 2),
        in_specs=[
            pl.BlockSpec((blk, D), lambda i, j: (i, 0)),
            pl.BlockSpec((D, H), lambda i, j: (0, j)),
            pl.BlockSpec((D, H), lambda i, j: (0, j)),
        ],
        out_specs=[
            pl.BlockSpec((blk, H), lambda i, j: (j * nb + i, 0)),
            pl.BlockSpec((blk, H), lambda i, j: (j * nb + i, 0)),
        ],
        out_shape=[
            jax.ShapeDtypeStruct((2 * NP, H), jnp.float32),
            jax.ShapeDtypeStruct((2 * NP, H), jnp.float32),
        ],
    )(xpad, Wl, Wr)


def _logits_body(xl_hbm, xr_hbm, src_hbm, src2_hbm, dst_hbm, dst2_hbm,
                 att_hbm, zn_hbm, a_hbm, den_hbm,
                 src_v, src2_v, dst_v, dst2_v, att_v, a_t, den_v, red_v,
                 xll, xlh, xrl, xrh, den_sh, sem):
    c = lax.axis_index("c")
    s = lax.axis_index("s")
    wid = s * NC + c

    pltpu.sync_copy(src_hbm.at[wid], src_v)
    pltpu.sync_copy(src2_hbm.at[wid], src2_v)
    pltpu.sync_copy(dst_hbm.at[wid], dst_v)
    pltpu.sync_copy(dst2_hbm.at[wid], dst2_v)
    pltpu.sync_copy(att_hbm, att_v)
    pltpu.sync_copy(zn_hbm, den_v)

    evs = [jnp.arange(L, dtype=jnp.int32) + g * L for g in range(4)]
    lanes = jnp.arange(L, dtype=jnp.int32)
    zero = jnp.zeros((L,), jnp.float32)

    def batch(b, _):
        cp0 = pltpu.make_async_copy(xl_hbm.at[src_v.at[b]], xll, sem)
        cp1 = pltpu.make_async_copy(xl_hbm.at[src2_v.at[b]], xlh, sem)
        cp2 = pltpu.make_async_copy(xr_hbm.at[dst_v.at[b]], xrl, sem)
        cp3 = pltpu.make_async_copy(xr_hbm.at[dst2_v.at[b]], xrh, sem)
        cp0.start(); cp1.start(); cp2.start(); cp3.start()
        cp0.wait(); cp1.wait(); cp2.wait(); cp3.wait()

        def dims(d, accs):
            dl = jnp.full((L,), d, jnp.int32)
            a_lo = att_v[pl.ds(d, L)][0]
            a_hi = att_v[pl.ds(d + H, L)][0]
            out = []
            for g in range(4):
                vl = plsc.load_gather(xll, [evs[g], dl])
                vr = plsc.load_gather(xrl, [evs[g], dl])
                t = vl + vr
                acc = accs[g] + a_lo * jnp.maximum(t, 0.2 * t)
                vl = plsc.load_gather(xlh, [evs[g], dl])
                vr = plsc.load_gather(xrh, [evs[g], dl])
                t = vl + vr
                acc = acc + a_hi * jnp.maximum(t, 0.2 * t)
                out.append(acc)
            return tuple(out)

        accs = lax.fori_loop(0, H, dims, (zero, zero, zero, zero))
        for g in range(4):
            ag = jnp.exp(accs[g])
            a_t[b, pl.ds(g * L, L)] = ag
            dg = dst_v[b, pl.ds(g * L, L)]
            for j in range(L):
                plsc.addupdate_scatter(den_v, [dg], ag, mask=lanes == j)
        return 0

    lax.fori_loop(0, NB32, batch, 0)
    pltpu.sync_copy(a_t, a_hbm.at[wid])

    pltpu.sync_copy(den_v, den_sh.at[s])
    plsc.subcore_barrier()
    for j in range(NS):
        pltpu.sync_copy(den_sh.at[j].at[pl.ds(s * RPT, RPT)], red_v.at[j])

    def red(k, _):
        acc = red_v[0, pl.ds(k * L, L)]
        for j in range(1, NS):
            acc = acc + red_v[j, pl.ds(k * L, L)]
        den_v[pl.ds(k * L, L)] = acc
        return 0

    lax.fori_loop(0, RPT // L, red, 0)
    pltpu.sync_copy(den_v.at[pl.ds(0, RPT)], den_hbm.at[c].at[pl.ds(s * RPT, RPT)])


def _edge_logits(xl_cat, xr_cat, src, src2, dst, dst2, att, zn):
    mesh = plsc.VectorSubcoreMesh(core_axis_name="c", subcore_axis_name="s")
    fn = pl.kernel(
        _logits_body,
        out_type=[
            jax.ShapeDtypeStruct((NC * NS, NB32, BT), jnp.float32),
            jax.ShapeDtypeStruct((NC, NP), jnp.float32),
        ],
        mesh=mesh,
        compiler_params=pltpu.CompilerParams(needs_layout_passes=False),
        scratch_types=[
            pltpu.VMEM((NB32, BT), jnp.int32),
            pltpu.VMEM((NB32, BT), jnp.int32),
            pltpu.VMEM((NB32, BT), jnp.int32),
            pltpu.VMEM((NB32, BT), jnp.int32),
            pltpu.VMEM((D + L,), jnp.float32),
            pltpu.VMEM((NB32, BT), jnp.float32),
            pltpu.VMEM((NP,), jnp.float32),
            pltpu.VMEM((NS, RPT), jnp.float32),
            pltpu.VMEM((BT, H), jnp.float32),
            pltpu.VMEM((BT, H), jnp.float32),
            pltpu.VMEM((BT, H), jnp.float32),
            pltpu.VMEM((BT, H), jnp.float32),
            pltpu.VMEM_SHARED((NS, NP), jnp.float32),
            pltpu.SemaphoreType.DMA,
        ],
    )
    return fn(xl_cat, xr_cat, src, src2, dst, dst2, att, zn)


def _scatter_body(xl_hbm, src_hbm, src2_hbm, dst_hbm, a_hbm, z_hbm, out_hbm,
                  srcc_v, dst_v, a_v, idx_v, gbuf, cbuf, osh, sem):
    c = lax.axis_index("c")
    s = lax.axis_index("s")

    @pl.when(c == 0)
    def _():
        pltpu.sync_copy(src_hbm.at[s], srcc_v)

    @pl.when(c == 1)
    def _():
        pltpu.sync_copy(src2_hbm.at[s], srcc_v)

    pltpu.sync_copy(dst_hbm.at[s], dst_v)
    pltpu.sync_copy(a_hbm.at[s], a_v)

    for p in range(2):
        lo = p * HNP
        pltpu.sync_copy(z_hbm, osh.at[pl.ds(s * (HNP // NS), HNP // NS)])
        plsc.subcore_barrier()

        def batch(b, _):
            pltpu.make_async_copy(xl_hbm.at[srcc_v.at[b]], gbuf, sem).start()
            pltpu.make_async_copy(xl_hbm.at[srcc_v.at[b]], gbuf, sem).wait()
            bl = jnp.full((L,), b, jnp.int32)

            for g in range(4):
                dg = dst_v[b, pl.ds(g * L, L)] - lo
                inr = (dg >= 0) & (dg < HNP)
                idx_v[pl.ds(g * L, L)] = jnp.where(inr, dg, HNP)

            def edge(e, _):
                ae = plsc.load_gather(a_v, [bl, jnp.full((L,), e, jnp.int32)])
                for k in range(H // L):
                    cbuf[e, pl.ds(k * L, L)] = gbuf[e, pl.ds(k * L, L)] * ae
                return 0

            lax.fori_loop(0, BT, edge, 0)
            pltpu.sync_copy(cbuf, osh.at[idx_v], add=True)
            return 0

        lax.fori_loop(0, NB16, batch, 0)
        plsc.subcore_barrier()
        stride = HNP // NS
        pltpu.sync_copy(
            osh.at[pl.ds(s * stride, stride)],
            out_hbm.at[c].at[pl.ds(lo + s * stride, stride)])
        plsc.subcore_barrier()


def _scatter(xl_cat, src16, src216, dst16, a16, zrows):
    mesh = plsc.VectorSubcoreMesh(core_axis_name="c", subcore_axis_name="s")
    fn = pl.kernel(
        _scatter_body,
        out_type=jax.ShapeDtypeStruct((NC, NP, H), jnp.float32),
        mesh=mesh,
        compiler_params=pltpu.CompilerParams(needs_layout_passes=False),
        scratch_types=[
            pltpu.VMEM((NB16, BT), jnp.int32),
            pltpu.VMEM((NB16, BT), jnp.int32),
            pltpu.VMEM((NB16, BT), jnp.float32),
            pltpu.VMEM((BT,), jnp.int32),
            pltpu.VMEM((BT, H), jnp.float32),
            pltpu.VMEM((BT, H), jnp.float32),
            pltpu.VMEM_SHARED((HNP + 8, H), jnp.float32),
            pltpu.SemaphoreType.DMA,
        ],
    )
    return fn(xl_cat, src16, src216, dst16, a16, zrows)


def _finish_body(pre_ref, den_ref, bias_ref, g_ref, b_ref, y_ref):
    p = pre_ref[...]
    f = jnp.concatenate([p[0], p[1]], axis=-1)
    dd = den_ref[...]
    den = (dd[0] + dd[1] + 1e-16)[:, None]
    o = f / den + bias_ref[...]
    mu = jnp.mean(o, axis=-1, keepdims=True)
    var = jnp.mean((o - mu) ** 2, axis=-1, keepdims=True)
    h = (o - mu) / jnp.sqrt(var + 1e-5) * g_ref[...] + b_ref[...]
    y_ref[...] = jnp.maximum(h, 0.0)


def _finish(pre, den, bias, gamma, beta):
    blk = 1024
    nb = NP // blk
    return pl.pallas_call(
        _finish_body,
        grid=(nb,),
        in_specs=[
            pl.BlockSpec((NC, blk, H), lambda i: (0, i, 0)),
            pl.BlockSpec((NC, blk), lambda i: (0, i)),
            pl.BlockSpec((1, D), lambda i: (0, 0)),
            pl.BlockSpec((1, D), lambda i: (0, 0)),
            pl.BlockSpec((1, D), lambda i: (0, 0)),
        ],
        out_specs=pl.BlockSpec((blk, D), lambda i: (i, 0)),
        out_shape=jax.ShapeDtypeStruct((NP, D), jnp.float32),
    )(pre, den, bias, gamma, beta)


@jax.jit
def kernel(x, edge_index, Wl, Wr, att, bias, ln_gamma, ln_beta):
    xpad = jnp.zeros((NP, D), jnp.float32).at[:N].set(x)

    loops = jnp.arange(N, dtype=jnp.int32)
    src = jnp.concatenate([edge_index[0].astype(jnp.int32), loops,
                           jnp.zeros((EP - ET,), jnp.int32)])
    dst = jnp.concatenate([edge_index[1].astype(jnp.int32), loops,
                           jnp.full((EP - ET,), N, jnp.int32)])
    src2 = src + NP
    dst2 = dst + NP

    xl_cat, xr_cat = _project(xpad, Wl, Wr)

    attp = jnp.concatenate([att, jnp.zeros((L,), jnp.float32)])
    zn = jnp.zeros((NP,), jnp.float32)
    r32 = lambda v: v.reshape(NC * NS, NB32, BT)
    a, den = _edge_logits(xl_cat, xr_cat, r32(src), r32(src2), r32(dst),
                          r32(dst2), attp, zn)

    r16 = lambda v: v.reshape(NS, NB16, BT)
    zrows = jnp.zeros((HNP // NS, H), jnp.float32)
    pre = _scatter(xl_cat, r16(src), r16(src2), r16(dst),
                   a.reshape(NS, NB16, BT), zrows)

    y = _finish(pre, den, bias.reshape(1, D), ln_gamma.reshape(1, D),
                ln_beta.reshape(1, D))
    return y[:N]


# speedup vs baseline: 1.6594x; 1.2496x over previous
import jax
import jax.numpy as jnp
from jax import lax
from jax.experimental import pallas as pl
from jax.experimental.pallas import tpu as pltpu
from jax.experimental.pallas import tpu_sc as plsc

N = 10000
NP = 10240
D = 256
H = 128
E = 160000
ET = E + N
EP = 172032
NC, NS, L = 2, 16, 16
NBB = 168
BTB = 32
NB16 = 168
BT = 64
HNP = NP // 2


def _proj_body(x_ref, wl_ref, wr_ref, xl_ref, xr_ref):
    xb = x_ref[...]
    xl_ref[...] = jnp.dot(xb, wl_ref[...], preferred_element_type=jnp.float32)
    xr_ref[...] = jnp.dot(xb, wr_ref[...], preferred_element_type=jnp.float32)


def _project(xpad, Wl, Wr):
    blk = 1024
    nb = NP // blk
    return pl.pallas_call(
        _proj_body,
        grid=(nb, 2),
        in_specs=[
            pl.BlockSpec((blk, D), lambda i, j: (i, 0)),
            pl.BlockSpec((D, H), lambda i, j: (0, j)),
            pl.BlockSpec((D, H), lambda i, j: (0, j)),
        ],
        out_specs=[
            pl.BlockSpec((blk, H), lambda i, j: (j * nb + i, 0)),
            pl.BlockSpec((blk, H), lambda i, j: (j * nb + i, 0)),
        ],
        out_shape=[
            jax.ShapeDtypeStruct((2 * NP, H), jnp.float32),
            jax.ShapeDtypeStruct((2 * NP, H), jnp.float32),
        ],
    )(xpad, Wl, Wr)


def _logits_body(xl_hbm, xr_hbm, src_hbm, src2_hbm, dst_hbm, dst2_hbm,
                 att_hbm, a_hbm, den_hbm,
                 src_v, src2_v, dst_v, dst2_v, att_v, ab0, ab1, den_v,
                 xll0, xlh0, xrl0, xrh0, xll1, xlh1, xrl1, xrh1,
                 sem0, sem1, wsm0, wsm1):
    c = lax.axis_index("c")
    s = lax.axis_index("s")
    wid = s * NC + c

    pltpu.sync_copy(src_hbm.at[wid], src_v)
    pltpu.sync_copy(src2_hbm.at[wid], src2_v)
    pltpu.sync_copy(dst_hbm.at[wid], dst_v)
    pltpu.sync_copy(dst2_hbm.at[wid], dst2_v)
    pltpu.sync_copy(att_hbm, att_v)

    evs = [jnp.arange(L, dtype=jnp.int32) + g * L for g in range(2)]
    lanes = jnp.arange(L, dtype=jnp.int32)
    zero = jnp.zeros((L,), jnp.float32)
    bufs = ((xll0, xlh0, xrl0, xrh0, sem0, ab0, wsm0),
            (xll1, xlh1, xrl1, xrh1, sem1, ab1, wsm1))

    def zden(k, _):
        den_v[pl.ds(k * L, L)] = zero
        return 0

    lax.fori_loop(0, NP // L, zden, 0, unroll=4)

    def cps(b, bset):
        bl, bh, rl, rh, sm = bset[:5]
        return (pltpu.make_async_copy(xl_hbm.at[src_v.at[b]], bl, sm),
                pltpu.make_async_copy(xl_hbm.at[src2_v.at[b]], bh, sm),
                pltpu.make_async_copy(xr_hbm.at[dst_v.at[b]], rl, sm),
                pltpu.make_async_copy(xr_hbm.at[dst2_v.at[b]], rh, sm))

    def compute(b, bset):
        bl, bh, rl, rh, _, ab, wsm = bset

        @pl.when(b >= 2)
        def _():
            pltpu.make_async_copy(ab, a_hbm.at[wid].at[b - 2], wsm).wait()

        def dims(d, accs):
            dl = jnp.full((L,), d, jnp.int32)
            a_lo = att_v[pl.ds(d, L)][0]
            a_hi = att_v[pl.ds(d + H, L)][0]
            out = []
            for g in range(2):
                vl = plsc.load_gather(bl, [evs[g], dl])
                vr = plsc.load_gather(rl, [evs[g], dl])
                t = vl + vr
                acc = accs[g] + a_lo * jnp.maximum(t, 0.2 * t)
                vl = plsc.load_gather(bh, [evs[g], dl])
                vr = plsc.load_gather(rh, [evs[g], dl])
                t = vl + vr
                acc = acc + a_hi * jnp.maximum(t, 0.2 * t)
                out.append(acc)
            return tuple(out)

        accs = lax.fori_loop(0, H, dims, (zero, zero), unroll=2)
        for g in range(2):
            ag = jnp.exp(accs[g])
            ab[pl.ds(g * L, L)] = ag
            dg = dst_v[b, pl.ds(g * L, L)]
            for j in range(L):
                plsc.addupdate_scatter(den_v, [dg], ag, mask=lanes == j)
        pltpu.make_async_copy(ab, a_hbm.at[wid].at[b], wsm).start()

    for cp in cps(0, bufs[0]):
        cp.start()

    def pair(i, _):
        b0 = 2 * i
        b1 = b0 + 1
        for cp in cps(b1, bufs[1]):
            cp.start()
        for cp in cps(b0, bufs[0]):
            cp.wait()
        compute(b0, bufs[0])

        @pl.when(b1 + 1 < NBB)
        def _():
            for cp in cps(b1 + 1, bufs[0]):
                cp.start()

        for cp in cps(b1, bufs[1]):
            cp.wait()
        compute(b1, bufs[1])
        return 0

    lax.fori_loop(0, NBB // 2, pair, 0)
    pltpu.make_async_copy(ab0, a_hbm.at[wid].at[NBB - 2], wsm0).wait()
    pltpu.make_async_copy(ab1, a_hbm.at[wid].at[NBB - 1], wsm1).wait()
    pltpu.sync_copy(den_v, den_hbm.at[wid])


def _edge_logits(xl_cat, xr_cat, src, src2, dst, dst2, att):
    mesh = plsc.VectorSubcoreMesh(core_axis_name="c", subcore_axis_name="s")
    fn = pl.kernel(
        _logits_body,
        out_type=[
            jax.ShapeDtypeStruct((NC * NS, NBB, BTB), jnp.float32),
            jax.ShapeDtypeStruct((NC * NS, NP), jnp.float32),
        ],
        mesh=mesh,
        compiler_params=pltpu.CompilerParams(needs_layout_passes=False),
        scratch_types=[
            pltpu.VMEM((NBB, BTB), jnp.int32),
            pltpu.VMEM((NBB, BTB), jnp.int32),
            pltpu.VMEM((NBB, BTB), jnp.int32),
            pltpu.VMEM((NBB, BTB), jnp.int32),
            pltpu.VMEM((D + L,), jnp.float32),
            pltpu.VMEM((BTB,), jnp.float32),
            pltpu.VMEM((BTB,), jnp.float32),
            pltpu.VMEM((NP,), jnp.float32),
            pltpu.VMEM((BTB, H), jnp.float32),
            pltpu.VMEM((BTB, H), jnp.float32),
            pltpu.VMEM((BTB, H), jnp.float32),
            pltpu.VMEM((BTB, H), jnp.float32),
            pltpu.VMEM((BTB, H), jnp.float32),
            pltpu.VMEM((BTB, H), jnp.float32),
            pltpu.VMEM((BTB, H), jnp.float32),
            pltpu.VMEM((BTB, H), jnp.float32),
            pltpu.SemaphoreType.DMA,
            pltpu.SemaphoreType.DMA,
            pltpu.SemaphoreType.DMA,
            pltpu.SemaphoreType.DMA,
        ],
    )
    return fn(xl_cat, xr_cat, src, src2, dst, dst2, att)


def _scatter_body(xl_hbm, src_hbm, dst_hbm, a_hbm, z_hbm, out_hbm,
                  srcc_v, a_v, dv0, dv1, idx0, idx1, gbuf0, gbuf1, cbuf0,
                  osh, gsm0, gsm1):
    c = lax.axis_index("c")
    s = lax.axis_index("s")

    pltpu.sync_copy(src_hbm.at[s], srcc_v)
    pltpu.sync_copy(a_hbm.at[s], a_v)

    coff = c * NP

    def shift(r, _):
        for g in range(4):
            sl = pl.ds(g * L, L)
            srcc_v[r, sl] = srcc_v[r, sl] + coff
        return 0

    lax.fori_loop(0, NB16, shift, 0, unroll=4)

    sets = ((dv0, idx0, gbuf0, gsm0), (dv1, idx1, gbuf1, gsm1))

    def gcp(b, bset):
        return (pltpu.make_async_copy(xl_hbm.at[srcc_v.at[b]], bset[2],
                                      bset[3]),
                pltpu.make_async_copy(dst_hbm.at[s].at[b], bset[0], bset[3]))

    for p in range(2):
        lo = p * HNP
        pltpu.sync_copy(z_hbm, osh.at[pl.ds(s * (HNP // NS), HNP // NS)])
        plsc.subcore_barrier()

        def work(b, bset):
            dv, idx_v, gbuf, _ = bset
            bl = jnp.full((L,), b, jnp.int32)
            for g in range(4):
                dg = dv[pl.ds(g * L, L)] - lo
                inr = (dg >= 0) & (dg < HNP)
                idx_v[pl.ds(g * L, L)] = jnp.where(inr, dg, HNP)

            def edge(e, _):
                ae = plsc.load_gather(a_v, [bl, jnp.full((L,), e, jnp.int32)])
                for k in range(H // L):
                    cbuf0[e, pl.ds(k * L, L)] = gbuf[e, pl.ds(k * L, L)] * ae
                return 0

            lax.fori_loop(0, BT, edge, 0, unroll=2)
            pltpu.sync_copy(cbuf0, osh.at[idx_v], add=True)

        for cp in gcp(0, sets[0]):
            cp.start()

        def pair(i, _):
            b0 = 2 * i
            b1 = b0 + 1
            for cp in gcp(b1, sets[1]):
                cp.start()
            for cp in gcp(b0, sets[0]):
                cp.wait()
            work(b0, sets[0])

            @pl.when(b1 + 1 < NB16)
            def _():
                for cp in gcp(b1 + 1, sets[0]):
                    cp.start()

            for cp in gcp(b1, sets[1]):
                cp.wait()
            work(b1, sets[1])
            return 0

        lax.fori_loop(0, NB16 // 2, pair, 0)
        plsc.subcore_barrier()
        stride = HNP // NS
        pltpu.sync_copy(
            osh.at[pl.ds(s * stride, stride)],
            out_hbm.at[c].at[pl.ds(lo + s * stride, stride)])
        plsc.subcore_barrier()


def _scatter(xl_cat, src16, dst16, a16, zrows):
    mesh = plsc.VectorSubcoreMesh(core_axis_name="c", subcore_axis_name="s")
    fn = pl.kernel(
        _scatter_body,
        out_type=jax.ShapeDtypeStruct((NC, NP, H), jnp.float32),
        mesh=mesh,
        compiler_params=pltpu.CompilerParams(needs_layout_passes=False),
        scratch_types=[
            pltpu.VMEM((NB16, BT), jnp.int32),
            pltpu.VMEM((NB16, BT), jnp.float32),
            pltpu.VMEM((BT,), jnp.int32),
            pltpu.VMEM((BT,), jnp.int32),
            pltpu.VMEM((BT,), jnp.int32),
            pltpu.VMEM((BT,), jnp.int32),
            pltpu.VMEM((BT, H), jnp.float32),
            pltpu.VMEM((BT, H), jnp.float32),
            pltpu.VMEM((BT, H), jnp.float32),
            pltpu.VMEM_SHARED((HNP + 8, H), jnp.float32),
            pltpu.SemaphoreType.DMA,
            pltpu.SemaphoreType.DMA,
        ],
    )
    return fn(xl_cat, src16, dst16, a16, zrows)


def _finish_body(pre_ref, den_ref, bias_ref, g_ref, b_ref, y_ref):
    p = pre_ref[...]
    f = jnp.concatenate([p[0], p[1]], axis=-1)
    den = (jnp.sum(den_ref[...], axis=0) + 1e-16)[:, None]
    o = f / den + bias_ref[...]
    mu = jnp.mean(o, axis=-1, keepdims=True)
    var = jnp.mean((o - mu) ** 2, axis=-1, keepdims=True)
    h = (o - mu) / jnp.sqrt(var + 1e-5) * g_ref[...] + b_ref[...]
    y_ref[...] = jnp.maximum(h, 0.0)


def _finish(pre, den, bias, gamma, beta):
    blk = 1024
    nb = NP // blk
    return pl.pallas_call(
        _finish_body,
        grid=(nb,),
        in_specs=[
            pl.BlockSpec((NC, blk, H), lambda i: (0, i, 0)),
            pl.BlockSpec((NC * NS, blk), lambda i: (0, i)),
            pl.BlockSpec((1, D), lambda i: (0, 0)),
            pl.BlockSpec((1, D), lambda i: (0, 0)),
            pl.BlockSpec((1, D), lambda i: (0, 0)),
        ],
        out_specs=pl.BlockSpec((blk, D), lambda i: (i, 0)),
        out_shape=jax.ShapeDtypeStruct((NP, D), jnp.float32),
    )(pre, den, bias, gamma, beta)


@jax.jit
def kernel(x, edge_index, Wl, Wr, att, bias, ln_gamma, ln_beta):
    xpad = jnp.zeros((NP, D), jnp.float32).at[:N].set(x)

    loops = jnp.arange(N, dtype=jnp.int32)
    src = jnp.concatenate([edge_index[0].astype(jnp.int32), loops,
                           jnp.zeros((EP - ET,), jnp.int32)])
    dst = jnp.concatenate([edge_index[1].astype(jnp.int32), loops,
                           jnp.full((EP - ET,), N, jnp.int32)])
    src2 = src + NP
    dst2 = dst + NP

    xl_cat, xr_cat = _project(xpad, Wl, Wr)

    attp = jnp.concatenate([att, jnp.zeros((L,), jnp.float32)])
    r32 = lambda v: v.reshape(NC * NS, NBB, BTB)
    a, den = _edge_logits(xl_cat, xr_cat, r32(src), r32(src2), r32(dst),
                          r32(dst2), attp)

    r16 = lambda v: v.reshape(NS, NB16, BT)
    zrows = jnp.zeros((HNP // NS, H), jnp.float32)
    pre = _scatter(xl_cat, r16(src), r16(dst), a.reshape(NS, NB16, BT), zrows)

    y = _finish(pre, den, bias.reshape(1, D), ln_gamma.reshape(1, D),
                ln_beta.reshape(1, D))
    return y[:N]


# speedup vs baseline: 3.1163x; 1.8780x over previous
import jax
import jax.numpy as jnp
from jax import lax
from jax.experimental import pallas as pl
from jax.experimental.pallas import tpu as pltpu
from jax.experimental.pallas import tpu_sc as plsc

N = 10000
NP = 10240
D = 256
H = 128
E = 160000
ET = E + N
EP = 172032
NC, NS, L = 2, 16, 16
NBB = 168
BTB = 32
NB16 = 168
BT = 64
HNP = NP // 2


def _proj_body(x_ref, wl_ref, wr_ref, xl_ref, xr_ref):
    xb = x_ref[...]
    xl_ref[...] = jnp.dot(xb, wl_ref[...], preferred_element_type=jnp.float32)
    xr_ref[...] = jnp.dot(xb, wr_ref[...], preferred_element_type=jnp.float32)


def _project(xpad, Wl, Wr):
    blk = 1024
    nb = NP // blk
    return pl.pallas_call(
        _proj_body,
        grid=(nb, 2),
        in_specs=[
            pl.BlockSpec((blk, D), lambda i, j: (i, 0)),
            pl.BlockSpec((D, H), lambda i, j: (0, j)),
            pl.BlockSpec((D, H), lambda i, j: (0, j)),
        ],
        out_specs=[
            pl.BlockSpec((blk, H), lambda i, j: (j * nb + i, 0)),
            pl.BlockSpec((blk, H), lambda i, j: (j * nb + i, 0)),
        ],
        out_shape=[
            jax.ShapeDtypeStruct((2 * NP, H), jnp.float32),
            jax.ShapeDtypeStruct((2 * NP, H), jnp.float32),
        ],
    )(xpad, Wl, Wr)


def _logits_body(xl_hbm, xr_hbm, src_hbm, src2_hbm, dst_hbm, dst2_hbm,
                 att_hbm, a_hbm, den_hbm,
                 src_v, src2_v, dst_v, dst2_v, att_v, ab0, ab1, den_v,
                 xll0, xlh0, xrl0, xrh0, xll1, xlh1, xrl1, xrh1,
                 sem0, sem1, wsm0, wsm1):
    c = lax.axis_index("c")
    s = lax.axis_index("s")
    wid = s * NC + c

    pltpu.sync_copy(src_hbm.at[wid], src_v)
    pltpu.sync_copy(src2_hbm.at[wid], src2_v)
    pltpu.sync_copy(dst_hbm.at[wid], dst_v)
    pltpu.sync_copy(dst2_hbm.at[wid], dst2_v)
    pltpu.sync_copy(att_hbm, att_v)

    lanes = jnp.arange(L, dtype=jnp.int32)
    zero = jnp.zeros((L,), jnp.float32)
    bufs = ((xll0, xlh0, xrl0, xrh0, sem0, ab0, wsm0),
            (xll1, xlh1, xrl1, xrh1, sem1, ab1, wsm1))

    def zden(k, _):
        den_v[pl.ds(k * L, L)] = zero
        return 0

    lax.fori_loop(0, NP // L, zden, 0, unroll=4)

    def cps(b, bset):
        bl, bh, rl, rh, sm = bset[:5]
        r = b // 4
        col = (b % 4) * BTB
        return (
            pltpu.make_async_copy(xl_hbm.at[src_v.at[r, pl.ds(col, BTB)]],
                                  bl, sm),
            pltpu.make_async_copy(xl_hbm.at[src2_v.at[r, pl.ds(col, BTB)]],
                                  bh, sm),
            pltpu.make_async_copy(xr_hbm.at[dst_v.at[r, pl.ds(col, BTB)]],
                                  rl, sm),
            pltpu.make_async_copy(xr_hbm.at[dst2_v.at[r, pl.ds(col, BTB)]],
                                  rh, sm))

    def compute(b, bset):
        bl, bh, rl, rh, _, ab, wsm = bset

        @pl.when(b >= 2)
        def _():
            pltpu.make_async_copy(ab, a_hbm.at[wid].at[b - 2], wsm).wait()

        attv = [att_v[pl.ds(k * L, L)] for k in range(D // L)]

        def grp(g, _):
            coll = zero
            for e16 in range(L):
                e = g * L + e16
                acc = zero
                for k in range(H // L):
                    sl = pl.ds(k * L, L)
                    t = bl[e, sl] + rl[e, sl]
                    acc = acc + attv[k] * jnp.maximum(t, 0.2 * t)
                    t = bh[e, sl] + rh[e, sl]
                    acc = acc + attv[8 + k] * jnp.maximum(t, 0.2 * t)
                coll = jnp.where(lanes == e16, jnp.sum(acc), coll)
            ag = jnp.exp(coll)
            ab[pl.ds(g * L, L)] = ag
            dg = dst_v[b // 4, pl.ds((b % 4) * BTB + g * L, L)]
            for j in range(L):
                plsc.addupdate_scatter(den_v, [dg], ag, mask=lanes == j)
            return 0

        lax.fori_loop(0, BTB // L, grp, 0)
        pltpu.make_async_copy(ab, a_hbm.at[wid].at[b], wsm).start()

    for cp in cps(0, bufs[0]):
        cp.start()

    def pair(i, _):
        b0 = 2 * i
        b1 = b0 + 1
        for cp in cps(b1, bufs[1]):
            cp.start()
        for cp in cps(b0, bufs[0]):
            cp.wait()
        compute(b0, bufs[0])

        @pl.when(b1 + 1 < NBB)
        def _():
            for cp in cps(b1 + 1, bufs[0]):
                cp.start()

        for cp in cps(b1, bufs[1]):
            cp.wait()
        compute(b1, bufs[1])
        return 0

    lax.fori_loop(0, NBB // 2, pair, 0)
    pltpu.make_async_copy(ab0, a_hbm.at[wid].at[NBB - 2], wsm0).wait()
    pltpu.make_async_copy(ab1, a_hbm.at[wid].at[NBB - 1], wsm1).wait()
    pltpu.sync_copy(den_v, den_hbm.at[wid])


def _edge_logits(xl_cat, xr_cat, src, src2, dst, dst2, att):
    mesh = plsc.VectorSubcoreMesh(core_axis_name="c", subcore_axis_name="s")
    fn = pl.kernel(
        _logits_body,
        out_type=[
            jax.ShapeDtypeStruct((NC * NS, NBB, BTB), jnp.float32),
            jax.ShapeDtypeStruct((NC * NS, NP), jnp.float32),
        ],
        mesh=mesh,
        compiler_params=pltpu.CompilerParams(needs_layout_passes=False),
        scratch_types=[
            pltpu.VMEM((NBB * BTB // 128, 128), jnp.int32),
            pltpu.VMEM((NBB * BTB // 128, 128), jnp.int32),
            pltpu.VMEM((NBB * BTB // 128, 128), jnp.int32),
            pltpu.VMEM((NBB * BTB // 128, 128), jnp.int32),
            pltpu.VMEM((D + L,), jnp.float32),
            pltpu.VMEM((BTB,), jnp.float32),
            pltpu.VMEM((BTB,), jnp.float32),
            pltpu.VMEM((NP,), jnp.float32),
            pltpu.VMEM((BTB, H), jnp.float32),
            pltpu.VMEM((BTB, H), jnp.float32),
            pltpu.VMEM((BTB, H), jnp.float32),
            pltpu.VMEM((BTB, H), jnp.float32),
            pltpu.VMEM((BTB, H), jnp.float32),
            pltpu.VMEM((BTB, H), jnp.float32),
            pltpu.VMEM((BTB, H), jnp.float32),
            pltpu.VMEM((BTB, H), jnp.float32),
            pltpu.SemaphoreType.DMA,
            pltpu.SemaphoreType.DMA,
            pltpu.SemaphoreType.DMA,
            pltpu.SemaphoreType.DMA,
        ],
    )
    return fn(xl_cat, xr_cat, src, src2, dst, dst2, att)


def _scatter_body(xl_hbm, src_hbm, dst_hbm, a_hbm, z_hbm, out_hbm,
                  srcc_v, a_v, dv0, dv1, idx0, idx1, gbuf0, gbuf1, cbuf0,
                  osh, gsm0, gsm1):
    c = lax.axis_index("c")
    s = lax.axis_index("s")

    pltpu.sync_copy(src_hbm.at[s], srcc_v)
    pltpu.sync_copy(a_hbm.at[s], a_v)

    coff = c * NP

    def shift(r, _):
        for g in range(4):
            sl = pl.ds(g * L, L)
            srcc_v[r, sl] = srcc_v[r, sl] + coff
        return 0

    lax.fori_loop(0, NB16, shift, 0, unroll=4)

    sets = ((dv0, idx0, gbuf0, gsm0), (dv1, idx1, gbuf1, gsm1))

    def gcp(b, bset):
        return (pltpu.make_async_copy(xl_hbm.at[srcc_v.at[b]], bset[2],
                                      bset[3]),
                pltpu.make_async_copy(dst_hbm.at[s].at[b], bset[0], bset[3]))

    for p in range(2):
        lo = p * HNP
        pltpu.sync_copy(z_hbm, osh.at[pl.ds(s * (HNP // NS), HNP // NS)])
        plsc.subcore_barrier()

        def work(b, bset):
            dv, idx_v, gbuf, _ = bset
            bl = jnp.full((L,), b, jnp.int32)
            for g in range(4):
                dg = dv[pl.ds(g * L, L)] - lo
                inr = (dg >= 0) & (dg < HNP)
                idx_v[pl.ds(g * L, L)] = jnp.where(inr, dg, HNP)

            def edge(e, _):
                ae = plsc.load_gather(a_v, [bl, jnp.full((L,), e, jnp.int32)])
                for k in range(H // L):
                    cbuf0[e, pl.ds(k * L, L)] = gbuf[e, pl.ds(k * L, L)] * ae
                return 0

            lax.fori_loop(0, BT, edge, 0, unroll=2)
            pltpu.sync_copy(cbuf0, osh.at[idx_v], add=True)

        for cp in gcp(0, sets[0]):
            cp.start()

        def pair(i, _):
            b0 = 2 * i
            b1 = b0 + 1
            for cp in gcp(b1, sets[1]):
                cp.start()
            for cp in gcp(b0, sets[0]):
                cp.wait()
            work(b0, sets[0])

            @pl.when(b1 + 1 < NB16)
            def _():
                for cp in gcp(b1 + 1, sets[0]):
                    cp.start()

            for cp in gcp(b1, sets[1]):
                cp.wait()
            work(b1, sets[1])
            return 0

        lax.fori_loop(0, NB16 // 2, pair, 0)
        plsc.subcore_barrier()
        stride = HNP // NS
        pltpu.sync_copy(
            osh.at[pl.ds(s * stride, stride)],
            out_hbm.at[c].at[pl.ds(lo + s * stride, stride)])
        plsc.subcore_barrier()


def _scatter(xl_cat, src16, dst16, a16, zrows):
    mesh = plsc.VectorSubcoreMesh(core_axis_name="c", subcore_axis_name="s")
    fn = pl.kernel(
        _scatter_body,
        out_type=jax.ShapeDtypeStruct((NC, NP, H), jnp.float32),
        mesh=mesh,
        compiler_params=pltpu.CompilerParams(needs_layout_passes=False),
        scratch_types=[
            pltpu.VMEM((NB16, BT), jnp.int32),
            pltpu.VMEM((NB16, BT), jnp.float32),
            pltpu.VMEM((BT,), jnp.int32),
            pltpu.VMEM((BT,), jnp.int32),
            pltpu.VMEM((BT,), jnp.int32),
            pltpu.VMEM((BT,), jnp.int32),
            pltpu.VMEM((BT, H), jnp.float32),
            pltpu.VMEM((BT, H), jnp.float32),
            pltpu.VMEM((BT, H), jnp.float32),
            pltpu.VMEM_SHARED((HNP + 8, H), jnp.float32),
            pltpu.SemaphoreType.DMA,
            pltpu.SemaphoreType.DMA,
        ],
    )
    return fn(xl_cat, src16, dst16, a16, zrows)


def _finish_body(pre_ref, den_ref, bias_ref, g_ref, b_ref, y_ref):
    p = pre_ref[...]
    f = jnp.concatenate([p[0], p[1]], axis=-1)
    den = (jnp.sum(den_ref[...], axis=0) + 1e-16)[:, None]
    o = f / den + bias_ref[...]
    mu = jnp.mean(o, axis=-1, keepdims=True)
    var = jnp.mean((o - mu) ** 2, axis=-1, keepdims=True)
    h = (o - mu) / jnp.sqrt(var + 1e-5) * g_ref[...] + b_ref[...]
    y_ref[...] = jnp.maximum(h, 0.0)


def _finish(pre, den, bias, gamma, beta):
    blk = 1024
    nb = NP // blk
    return pl.pallas_call(
        _finish_body,
        grid=(nb,),
        in_specs=[
            pl.BlockSpec((NC, blk, H), lambda i: (0, i, 0)),
            pl.BlockSpec((NC * NS, blk), lambda i: (0, i)),
            pl.BlockSpec((1, D), lambda i: (0, 0)),
            pl.BlockSpec((1, D), lambda i: (0, 0)),
            pl.BlockSpec((1, D), lambda i: (0, 0)),
        ],
        out_specs=pl.BlockSpec((blk, D), lambda i: (i, 0)),
        out_shape=jax.ShapeDtypeStruct((NP, D), jnp.float32),
    )(pre, den, bias, gamma, beta)


@jax.jit
def kernel(x, edge_index, Wl, Wr, att, bias, ln_gamma, ln_beta):
    xpad = jnp.zeros((NP, D), jnp.float32).at[:N].set(x)

    loops = jnp.arange(N, dtype=jnp.int32)
    src = jnp.concatenate([edge_index[0].astype(jnp.int32), loops,
                           jnp.zeros((EP - ET,), jnp.int32)])
    dst = jnp.concatenate([edge_index[1].astype(jnp.int32), loops,
                           jnp.full((EP - ET,), N, jnp.int32)])
    src2 = src + NP
    dst2 = dst + NP

    xl_cat, xr_cat = _project(xpad, Wl, Wr)

    attp = jnp.concatenate([att, jnp.zeros((L,), jnp.float32)])
    r32 = lambda v: v.reshape(NC * NS, NBB * BTB // 128, 128)
    a, den = _edge_logits(xl_cat, xr_cat, r32(src), r32(src2), r32(dst),
                          r32(dst2), attp)

    r16 = lambda v: v.reshape(NS, NB16, BT)
    zrows = jnp.zeros((HNP // NS, H), jnp.float32)
    pre = _scatter(xl_cat, r16(src), r16(dst), a.reshape(NS, NB16, BT), zrows)

    y = _finish(pre, den, bias.reshape(1, D), ln_gamma.reshape(1, D),
                ln_beta.reshape(1, D))
    return y[:N]
